# baseline (speedup 1.0000x reference)
import jax
import jax.numpy as jnp
from jax import lax
from jax.experimental import pallas as pl
from jax.experimental.pallas import tpu as pltpu
from jax.experimental.pallas import tpu_sc as plsc

_B, _S, _D = 4096, 200, 64
_NC, _NS = 2, 16
_NW = _NC * _NS
_RPW = _B // _NW
_SP = 208
_NPAIR = _RPW // 2


def _body(smiles_hbm, ads_hbm, stab_hbm, atab_hbm, ptab_hbm, out_hbm,
          pos_v, ads_idx_v, ads_rows_v,
          tok0, tok1, gath0, gath1, out0, out1,
          sem, semg0, semg1, semo0, semo1, semt0, semt1):
    cid = lax.axis_index("c")
    sid = lax.axis_index("s")
    wid = sid * _NC + cid
    row0 = wid * _RPW

    pltpu.sync_copy(ptab_hbm, pos_v)
    pltpu.sync_copy(ads_hbm.at[pl.ds(row0, _RPW)], ads_idx_v)
    pltpu.async_copy(atab_hbm.at[ads_idx_v], ads_rows_v, sem).wait()

    def tok_copy(r, tok, semt):
        return pltpu.async_copy(
            smiles_hbm.at[pl.ds((row0 + r) * _S, _S)], tok.at[pl.ds(0, _S)], semt)

    _CHUNKS = (0, 56, 112, 168)
    _CLENS = (56, 56, 56, 32)

    def gather(tok, gath, semg):
        for c0, cl in zip(_CHUNKS, _CLENS):
            pltpu.async_copy(
                stab_hbm.at[tok.at[pl.ds(c0, cl)]],
                gath.at[pl.ds(c0, cl)], semg)

    def gather_wait(tok, gath, semg):
        for c0, cl in zip(_CHUNKS, _CLENS):
            pltpu.make_async_copy(
                stab_hbm.at[tok.at[pl.ds(c0, cl)]],
                gath.at[pl.ds(c0, cl)], semg).wait()

    def compute(r, tok, gath, out):
        ads4 = tuple(ads_rows_v[r, pl.ds(c * 16, 16)] for c in range(4))

        def do_token(t, tv):
            m8 = jnp.where(tv != 0, jnp.float32(8.0), jnp.float32(0.0))
            m1 = jnp.where(tv != 0, jnp.float32(1.0), jnp.float32(0.0))
            m8v = lax.broadcast(m8, (16,))
            m1v = lax.broadcast(m1, (16,))
            for c2 in range(2):
                v = gath[t, pl.ds(c2 * 32, 32)]
                lo_hi = plsc.unpack(v, format=plsc.PackFormat.INTERLEAVED)
                for h in range(2):
                    c = c2 * 2 + h
                    p = pos_v[t, pl.ds(c * 16, 16)]
                    out[t, pl.ds(c * 16, 16)] = (
                        lo_hi[h] * m8v + (p + ads4[c]) * m1v)

        def grp_fn(gidx, inner_carry):
            t0 = gidx * 16
            tvec = tok[pl.ds(t0, 16)]
            for i in range(16):
                do_token(t0 + i, tvec[i])
            return inner_carry

        lax.fori_loop(0, _S // 16, grp_fn, 0)
        tvec_tail = tok[pl.ds(_S - _S % 16, 16)]
        for i in range(_S % 16):
            do_token(_S - _S % 16 + i, tvec_tail[i])

    tok_copy(0, tok0, semt0).wait()
    gather(tok0, gath0, semg0)
    tok_copy(1, tok1, semt1)

    def pair_fn(k, carry):
        a = 2 * k
        b = 2 * k + 1
        gather_wait(tok0, gath0, semg0)
        pltpu.make_async_copy(
            smiles_hbm.at[pl.ds((row0 + b) * _S, _S)], tok1.at[pl.ds(0, _S)],
            semt1).wait()
        gather(tok1, gath1, semg1)

        @pl.when(k > 0)
        def _():
            pltpu.make_async_copy(out0, out_hbm.at[row0 + a], semo0).wait()

        compute(a, tok0, gath0, out0)
        pltpu.async_copy(out0, out_hbm.at[row0 + a], semo0)

        @pl.when(k < _NPAIR - 1)
        def _():
            tok_copy(a + 2, tok0, semt0)

        gather_wait(tok1, gath1, semg1)

        @pl.when(k < _NPAIR - 1)
        def _():
            pltpu.make_async_copy(
                smiles_hbm.at[pl.ds((row0 + a + 2) * _S, _S)],
                tok0.at[pl.ds(0, _S)], semt0).wait()
            gather(tok0, gath0, semg0)

        @pl.when(k > 0)
        def _():
            pltpu.make_async_copy(out1, out_hbm.at[row0 + b], semo1).wait()

        compute(b, tok1, gath1, out1)
        pltpu.async_copy(out1, out_hbm.at[row0 + b], semo1)

        @pl.when(k < _NPAIR - 1)
        def _():
            tok_copy(b + 2, tok1, semt1)

        return carry

    lax.fori_loop(0, _NPAIR, pair_fn, 0)
    pltpu.make_async_copy(out0, out_hbm.at[row0 + _RPW - 2], semo0).wait()
    pltpu.make_async_copy(out1, out_hbm.at[row0 + _RPW - 1], semo1).wait()


def kernel(smiles, adsorbent, smile_table, ads_table, pos_table):
    smiles_flat = smiles.reshape(-1).astype(jnp.int32)
    ads = adsorbent.astype(jnp.int32)
    stab_bf = (smile_table.astype(jnp.bfloat16)
               .reshape(-1, 2, 2, 16).transpose(0, 1, 3, 2).reshape(-1, _D))
    mesh = plsc.VectorSubcoreMesh(core_axis_name="c", subcore_axis_name="s")
    out = pl.kernel(
        _body,
        out_type=jax.ShapeDtypeStruct((_B, _S, _D), jnp.float32),
        mesh=mesh,
        scratch_types=[
            pltpu.VMEM((_S, _D), jnp.float32),
            pltpu.VMEM((_RPW,), jnp.int32),
            pltpu.VMEM((_RPW, _D), jnp.float32),
            pltpu.VMEM((_SP,), jnp.int32),
            pltpu.VMEM((_SP,), jnp.int32),
            pltpu.VMEM((_SP, _D), jnp.bfloat16),
            pltpu.VMEM((_SP, _D), jnp.bfloat16),
            pltpu.VMEM((_S, _D), jnp.float32),
            pltpu.VMEM((_S, _D), jnp.float32),
            pltpu.SemaphoreType.DMA,
            pltpu.SemaphoreType.DMA,
            pltpu.SemaphoreType.DMA,
            pltpu.SemaphoreType.DMA,
            pltpu.SemaphoreType.DMA,
            pltpu.SemaphoreType.DMA,
            pltpu.SemaphoreType.DMA,
        ],
        compiler_params=pltpu.CompilerParams(
            use_tc_tiling_on_sc=False, needs_layout_passes=False),
    )(smiles_flat, ads, stab_bf, ads_table, pos_table)
    return out


# speedup vs baseline: 1.6327x; 1.6327x over previous
import jax
import jax.numpy as jnp
from jax import lax
from jax.experimental import pallas as pl
from jax.experimental.pallas import tpu as pltpu
from jax.experimental.pallas import tpu_sc as plsc

_B, _S, _D = 4096, 200, 64
_DP = 128
_NC, _NS = 2, 16
_NW = _NC * _NS
_RPW = _B // _NW
_SP = 208
_NPAIR = _RPW // 2


def _body(smiles_hbm, ads_hbm, stab_hbm, atab_hbm, ptab_hbm, out_hbm,
          pos_v, ads_idx_v, ads_rows_v,
          tok0, tok1, gath0, gath1, out_v,
          sem, semg0, semg1, semo, semt0, semt1):
    cid = lax.axis_index("c")
    sid = lax.axis_index("s")
    wid = sid * _NC + cid
    row0 = wid * _RPW

    pltpu.sync_copy(ptab_hbm, pos_v)
    pltpu.sync_copy(ads_hbm.at[pl.ds(row0, _RPW)], ads_idx_v)
    pltpu.async_copy(atab_hbm.at[ads_idx_v], ads_rows_v, sem).wait()

    def tok_copy(r, tok, semt):
        return pltpu.async_copy(
            smiles_hbm.at[pl.ds((row0 + r) * _S, _S)], tok.at[pl.ds(0, _S)], semt)

    _CHUNKS = (0, 56, 112, 168)
    _CLENS = (56, 56, 56, 32)

    def gather(tok, gath, semg):
        for c0, cl in zip(_CHUNKS, _CLENS):
            pltpu.async_copy(
                stab_hbm.at[tok.at[pl.ds(c0, cl)]],
                gath.at[pl.ds(c0, cl)], semg)

    def gather_wait(tok, gath, semg):
        for c0, cl in zip(_CHUNKS, _CLENS):
            pltpu.make_async_copy(
                stab_hbm.at[tok.at[pl.ds(c0, cl)]],
                gath.at[pl.ds(c0, cl)], semg).wait()

    def compute(r, tok, gath, out):
        ads4 = tuple(ads_rows_v[r, pl.ds(c * 16, 16)] for c in range(4))

        def do_token(t, tv):
            m8 = jnp.where(tv != 0, jnp.float32(8.0), jnp.float32(0.0))
            m1 = jnp.where(tv != 0, jnp.float32(1.0), jnp.float32(0.0))
            m8v = lax.broadcast(m8, (16,))
            m1v = lax.broadcast(m1, (16,))
            for c in range(4):
                g = gath[t, pl.ds(c * 16, 16)]
                p = pos_v[t, pl.ds(c * 16, 16)]
                out[t, pl.ds(c * 16, 16)] = g * m8v + (p + ads4[c]) * m1v

        def grp_fn(gidx, inner_carry):
            t0 = gidx * 16
            tvec = tok[pl.ds(t0, 16)]
            for i in range(16):
                do_token(t0 + i, tvec[i])
            return inner_carry

        lax.fori_loop(0, _S // 16, grp_fn, 0)
        tvec_tail = tok[pl.ds(_S - _S % 16, 16)]
        for i in range(_S % 16):
            do_token(_S - _S % 16 + i, tvec_tail[i])

    tok_copy(0, tok0, semt0).wait()
    gather(tok0, gath0, semg0)
    tok_copy(1, tok1, semt1)

    def pair_fn(k, carry):
        a = 2 * k
        b = 2 * k + 1
        gather_wait(tok0, gath0, semg0)
        pltpu.make_async_copy(
            smiles_hbm.at[pl.ds((row0 + b) * _S, _S)], tok1.at[pl.ds(0, _S)],
            semt1).wait()
        gather(tok1, gath1, semg1)

        @pl.when(k > 0)
        def _():
            pltpu.make_async_copy(out_v, out_hbm.at[row0 + a - 1], semo).wait()

        compute(a, tok0, gath0, out_v)
        pltpu.async_copy(out_v, out_hbm.at[row0 + a], semo)

        @pl.when(k < _NPAIR - 1)
        def _():
            tok_copy(a + 2, tok0, semt0)

        gather_wait(tok1, gath1, semg1)

        @pl.when(k < _NPAIR - 1)
        def _():
            pltpu.make_async_copy(
                smiles_hbm.at[pl.ds((row0 + a + 2) * _S, _S)],
                tok0.at[pl.ds(0, _S)], semt0).wait()
            gather(tok0, gath0, semg0)

        pltpu.make_async_copy(out_v, out_hbm.at[row0 + a], semo).wait()
        compute(b, tok1, gath1, out_v)
        pltpu.async_copy(out_v, out_hbm.at[row0 + b], semo)

        @pl.when(k < _NPAIR - 1)
        def _():
            tok_copy(b + 2, tok1, semt1)

        return carry

    lax.fori_loop(0, _NPAIR, pair_fn, 0)
    pltpu.make_async_copy(out_v, out_hbm.at[row0 + _RPW - 1], semo).wait()


def kernel(smiles, adsorbent, smile_table, ads_table, pos_table):
    smiles_flat = smiles.reshape(-1).astype(jnp.int32)
    ads = adsorbent.astype(jnp.int32)
    zpad = jnp.zeros((smile_table.shape[0], _DP - _D), jnp.float32)
    stab = jnp.concatenate([smile_table, zpad], axis=1)
    atab = jnp.concatenate(
        [ads_table, jnp.zeros((ads_table.shape[0], _DP - _D), jnp.float32)], axis=1)
    ptab = jnp.concatenate(
        [pos_table, jnp.zeros((pos_table.shape[0], _DP - _D), jnp.float32)], axis=1)
    mesh = plsc.VectorSubcoreMesh(core_axis_name="c", subcore_axis_name="s")
    out = pl.kernel(
        _body,
        out_type=jax.ShapeDtypeStruct((_B, _S, _D), jnp.float32),
        mesh=mesh,
        scratch_types=[
            pltpu.VMEM((_S, _DP), jnp.float32),
            pltpu.VMEM((_RPW,), jnp.int32),
            pltpu.VMEM((_RPW, _DP), jnp.float32),
            pltpu.VMEM((_SP,), jnp.int32),
            pltpu.VMEM((_SP,), jnp.int32),
            pltpu.VMEM((_SP, _DP), jnp.float32),
            pltpu.VMEM((_SP, _DP), jnp.float32),
            pltpu.VMEM((_S, _D), jnp.float32),
            pltpu.SemaphoreType.DMA,
            pltpu.SemaphoreType.DMA,
            pltpu.SemaphoreType.DMA,
            pltpu.SemaphoreType.DMA,
            pltpu.SemaphoreType.DMA,
            pltpu.SemaphoreType.DMA,
        ],
        compiler_params=pltpu.CompilerParams(use_tc_tiling_on_sc=True),
    )(smiles_flat, ads, stab, atab, ptab)
    return out
